# baseline (speedup 1.0000x reference)
import jax
import jax.numpy as jnp
from jax import lax
from jax.experimental import pallas as pl
from jax.experimental.pallas import tpu as pltpu
from jax.experimental.pallas import tpu_sc as plsc

_H = 32
_W = 32
_D = 256
_B = 8
_C = 2 * _D
_NC = 2
_NS = 16
_NW = _NC * _NS
_L = 16


_SC0_B = 6


def _fill_right_half(slab, row_v, row_idx, lanes):
    ii = jnp.full((_L,), row_idx, jnp.int32)
    for k in range(_D // _L):
        v = plsc.load_gather(row_v, [ii, lanes + k * _L])
        for j in range(_W):
            slab[j, pl.ds(_D + k * _L, _L)] = v


def _pos_body(row_hbm, col_hbm, out_hbm, row_v, slab_v, slab2_v,
              sem, sem_c, sem_r):
    cid = lax.axis_index("c")
    i = lax.axis_index("s") * _NC + cid
    c_col = pltpu.async_copy(col_hbm.at[pl.ds(0, _W)],
                             slab_v.at[:, pl.ds(0, _D)], sem_c)
    c_col2 = pltpu.async_copy(col_hbm.at[pl.ds(0, _W)],
                              slab2_v.at[:, pl.ds(0, _D)], sem_c)
    c_row = pltpu.async_copy(row_hbm.at[pl.ds(0, _H)], row_v, sem_r)
    c_row.wait()
    lanes = lax.iota(jnp.int32, _L)
    _fill_right_half(slab_v, row_v, i, lanes)
    c_col.wait()
    c_col2.wait()

    @pl.when(cid == 0)
    def _sc0_writes():
        copies = [pltpu.async_copy(slab_v, out_hbm.at[b, i], sem)
                  for b in range(_SC0_B)]
        for cp in copies:
            cp.wait()

    @pl.when(cid == 1)
    def _sc1_writes():
        _fill_right_half(slab2_v, row_v, i - 1, lanes)
        copies = [pltpu.async_copy(slab_v, out_hbm.at[b, i], sem)
                  for b in range(_B)]
        copies += [pltpu.async_copy(slab2_v, out_hbm.at[b, i - 1], sem)
                   for b in range(_SC0_B, _B)]
        for cp in copies:
            cp.wait()


@jax.jit
def _pos_embed(row_embed, col_embed):
    mesh = plsc.VectorSubcoreMesh(core_axis_name="c", subcore_axis_name="s")
    out = pl.kernel(
        _pos_body,
        out_type=jax.ShapeDtypeStruct((_B, _H, _W, _C), jnp.float32),
        mesh=mesh,
        scratch_types=[
            pltpu.VMEM((_H, _D), jnp.float32),
            pltpu.VMEM((_W, _C), jnp.float32),
            pltpu.VMEM((_W, _C), jnp.float32),
            pltpu.SemaphoreType.DMA,
            pltpu.SemaphoreType.DMA,
            pltpu.SemaphoreType.DMA,
        ],
        compiler_params=pltpu.CompilerParams(needs_layout_passes=False),
    )(row_embed, col_embed)
    return jnp.transpose(out, (0, 3, 1, 2))


def kernel(x, row_embed, col_embed):
    assert x.shape[0] == _B and x.shape[-2:] == (_H, _W)
    return _pos_embed(row_embed, col_embed)


# speedup vs baseline: 5.0706x; 5.0706x over previous
import jax
import jax.numpy as jnp
from jax.experimental import pallas as pl
from jax.experimental.pallas import tpu as pltpu

_H = 32
_W = 32
_D = 256
_B = 8
_C = 2 * _D


def _tc_body(row_ref, col_ref, out_ref, slab, sem):
    col = col_ref[0:_W, :]
    row = row_ref[0:_H, :]
    slab[:, :, 0:_D] = jnp.broadcast_to(col[None, :, :], (_H, _W, _D))
    slab[:, :, _D:_C] = jnp.broadcast_to(row[:, None, :], (_H, _W, _D))
    copies = [pltpu.make_async_copy(slab, out_ref.at[b], sem)
              for b in range(_B)]
    for cp in copies:
        cp.start()
    for cp in copies:
        cp.wait()


@jax.jit
def _pos_embed(row_embed, col_embed):
    out = pl.pallas_call(
        _tc_body,
        out_shape=jax.ShapeDtypeStruct((_B, _H, _W, _C), jnp.float32),
        in_specs=[
            pl.BlockSpec(memory_space=pltpu.VMEM),
            pl.BlockSpec(memory_space=pltpu.VMEM),
        ],
        out_specs=pl.BlockSpec(memory_space=pl.ANY),
        scratch_shapes=[
            pltpu.VMEM((_H, _W, _C), jnp.float32),
            pltpu.SemaphoreType.DMA,
        ],
    )(row_embed, col_embed)
    return jnp.transpose(out, (0, 3, 1, 2))


def kernel(x, row_embed, col_embed):
    assert x.shape[0] == _B and x.shape[-2:] == (_H, _W)
    return _pos_embed(row_embed, col_embed)
